# Initial kernel scaffold; baseline (speedup 1.0000x reference)
import functools

import jax
import jax.numpy as jnp
from jax import lax
from jax.experimental import pallas as pl
from jax.experimental.pallas import tpu as pltpu
from jax.experimental.pallas import tpu_sc as plsc

B = 10000
D = 256
K = 16
EMB = 256

NC = 2
NS = 16
NW = NC * NS
C = 64
NCH = 5
BPW = C * NCH
BP = NW * BPW
GRP = D // 16

_sc_mesh = plsc.VectorSubcoreMesh(core_axis_name="c", subcore_axis_name="s")


@functools.partial(
    pl.kernel,
    out_type=[
        jax.ShapeDtypeStruct((BP, D), jnp.float32),
        jax.ShapeDtypeStruct((BP, D), jnp.float32),
    ],
    mesh=_sc_mesh,
    scratch_types=[
        pltpu.VMEM((NCH, C), jnp.int32),
        pltpu.VMEM((K, NCH, C), jnp.int32),
        pltpu.VMEM((C, D), jnp.float32),
        pltpu.VMEM((2, C, D), jnp.float32),
        pltpu.VMEM((C, D), jnp.float32),
        pltpu.SemaphoreType.DMA,
        pltpu.SemaphoreType.DMA,
        pltpu.SemaphoreType.DMA,
        pltpu.SemaphoreType.DMA,
    ],
)
def _sc_gather(nodes_hbm, neigh_hbm, table_hbm, self_out, nsum_out,
               nodes_v, neigh_v, self_v, nbuf, acc,
               sem_a, sem_s, sem_n0, sem_n1):
    cid = lax.axis_index("c")
    sid = lax.axis_index("s")
    w = sid * NC + cid
    base = w * BPW

    pltpu.sync_copy(nodes_hbm.at[w], nodes_v)
    pltpu.sync_copy(neigh_hbm.at[w], neigh_v)

    def accum_col(buf_ref):
        def body(r, _):
            for g in range(GRP):
                sl = pl.ds(g * 16, 16)
                plsc.addupdate(acc.at[r, sl], buf_ref[r, sl])
            return 0
        lax.fori_loop(0, C, body, 0)

    for ch in range(NCH):
        row0 = base + ch * C
        ha = pltpu.async_copy(table_hbm.at[neigh_v.at[0, ch]], acc, sem_a)
        h0 = pltpu.async_copy(table_hbm.at[neigh_v.at[1, ch]], nbuf.at[0], sem_n0)
        h1 = pltpu.async_copy(table_hbm.at[neigh_v.at[2, ch]], nbuf.at[1], sem_n1)
        hs = pltpu.async_copy(table_hbm.at[nodes_v.at[ch]], self_v, sem_s)
        ha.wait()
        for j in range(1, K):
            slot = (j - 1) % 2
            (h0 if slot == 0 else h1).wait()
            accum_col(nbuf.at[slot])
            if j + 2 < K + 1:
                pass
            if j + 2 <= K - 1 + 1 and j + 2 < K + 1:
                pass
            if j + 2 < K + 1 and (j + 2) <= K:
                pass
            if j + 2 <= K:
                if j + 2 < K + 1:
                    pass
            if j + 2 < K + 1:
                pass
            if j + 2 <= K - 1:
                h = pltpu.async_copy(
                    table_hbm.at[neigh_v.at[j + 2, ch]], nbuf.at[slot],
                    sem_n0 if slot == 0 else sem_n1)
                if slot == 0:
                    h0 = h
                else:
                    h1 = h
        hs.wait()
        pltpu.sync_copy(self_v, self_out.at[pl.ds(row0, C)])
        pltpu.sync_copy(acc, nsum_out.at[pl.ds(row0, C)])


def _mm_body(x1_ref, x2_ref, w1_ref, w2_ref, b_ref, o_ref):
    acc = jnp.dot(x1_ref[...], w1_ref[...], preferred_element_type=jnp.float32)
    acc = acc + jnp.dot(x2_ref[...], w2_ref[...], preferred_element_type=jnp.float32)
    o_ref[...] = jnp.maximum(acc + b_ref[...], 0.0)


_BM = 1000


def _dense(x1, x2, w1, w2, b2d):
    return pl.pallas_call(
        _mm_body,
        grid=(B // _BM,),
        in_specs=[
            pl.BlockSpec((_BM, D), lambda i: (i, 0)),
            pl.BlockSpec((_BM, D), lambda i: (i, 0)),
            pl.BlockSpec((D, EMB), lambda i: (0, 0)),
            pl.BlockSpec((D, EMB), lambda i: (0, 0)),
            pl.BlockSpec((1, EMB), lambda i: (0, 0)),
        ],
        out_specs=pl.BlockSpec((_BM, EMB), lambda i: (i, 0)),
        out_shape=jax.ShapeDtypeStruct((B, EMB), jnp.float32),
    )(x1, x2, w1, w2, b2d)


def kernel(table, nodes, neigh_idx, W, b):
    nodes_i = nodes.astype(jnp.int32)
    neigh_i = neigh_idx.astype(jnp.int32)
    pad = BP - B
    nodes_p = jnp.concatenate([nodes_i, jnp.zeros((pad,), jnp.int32)])
    neigh_p = jnp.concatenate([neigh_i, jnp.zeros((pad, K), jnp.int32)])
    nodes_r = nodes_p.reshape(NW, NCH, C)
    neigh_r = jnp.transpose(neigh_p.reshape(NW, NCH, C, K), (0, 3, 1, 2))

    self_f, nsum = _sc_gather(nodes_r, neigh_r, table)

    w1 = W[:D]
    w2 = W[D:] * (1.0 / K)
    out = _dense(self_f[:B], nsum[:B], w1, w2, b.reshape(1, EMB))
    return out


# speedup vs baseline: 1.3536x; 1.3536x over previous
import functools

import jax
import jax.numpy as jnp
from jax import lax
from jax.experimental import pallas as pl
from jax.experimental.pallas import tpu as pltpu
from jax.experimental.pallas import tpu_sc as plsc

B = 10000
D = 256
K = 16
EMB = 256

NC = 2
NS = 16
NW = NC * NS
C = 64
NCH = 5
BPW = C * NCH
BP = NW * BPW
GRP = D // 16

_sc_mesh = plsc.VectorSubcoreMesh(core_axis_name="c", subcore_axis_name="s")


@functools.partial(
    pl.kernel,
    out_type=[
        jax.ShapeDtypeStruct((BP, D), jnp.float32),
        jax.ShapeDtypeStruct((BP, D), jnp.float32),
    ],
    mesh=_sc_mesh,
    scratch_types=[
        pltpu.VMEM((NCH, C), jnp.int32),
        pltpu.VMEM((K, NCH, C), jnp.int32),
        pltpu.VMEM((C, D), jnp.float32),
        pltpu.VMEM((2, C, D), jnp.float32),
        pltpu.VMEM((C, D), jnp.float32),
        pltpu.SemaphoreType.DMA,
        pltpu.SemaphoreType.DMA,
        pltpu.SemaphoreType.DMA,
        pltpu.SemaphoreType.DMA,
    ],
)
def _sc_gather(nodes_hbm, neigh_hbm, table_hbm, self_out, nsum_out,
               nodes_v, neigh_v, self_v, nbuf, acc,
               sem_a, sem_s, sem_n0, sem_n1):
    cid = lax.axis_index("c")
    sid = lax.axis_index("s")
    w = sid * NC + cid
    base = w * BPW

    pltpu.sync_copy(nodes_hbm.at[w], nodes_v)
    pltpu.sync_copy(neigh_hbm.at[w], neigh_v)

    def accum_col(buf_ref):
        def body(r, _):
            for g in range(GRP):
                sl = pl.ds(g * 16, 16)
                plsc.addupdate(acc.at[r, sl], buf_ref[r, sl])
            return 0
        lax.fori_loop(0, C, body, 0)

    def chunk_body(ch, _):
        row0 = base + ch * C
        ha = pltpu.async_copy(table_hbm.at[neigh_v.at[0, ch]], acc, sem_a)
        h0 = pltpu.async_copy(table_hbm.at[neigh_v.at[1, ch]], nbuf.at[0], sem_n0)
        h1 = pltpu.async_copy(table_hbm.at[neigh_v.at[2, ch]], nbuf.at[1], sem_n1)
        hs = pltpu.async_copy(table_hbm.at[nodes_v.at[ch]], self_v, sem_s)
        ha.wait()
        for j in range(1, K):
            slot = (j - 1) % 2
            (h0 if slot == 0 else h1).wait()
            accum_col(nbuf.at[slot])
            if j + 2 <= K - 1:
                h = pltpu.async_copy(
                    table_hbm.at[neigh_v.at[j + 2, ch]], nbuf.at[slot],
                    sem_n0 if slot == 0 else sem_n1)
                if slot == 0:
                    h0 = h
                else:
                    h1 = h
        hs.wait()
        pltpu.sync_copy(self_v, self_out.at[pl.ds(row0, C)])
        pltpu.sync_copy(acc, nsum_out.at[pl.ds(row0, C)])
        return 0

    lax.fori_loop(0, NCH, chunk_body, 0)


def _mm_body(x1_ref, x2_ref, w1_ref, w2_ref, b_ref, o_ref):
    acc = jnp.dot(x1_ref[...], w1_ref[...], preferred_element_type=jnp.float32)
    acc = acc + jnp.dot(x2_ref[...], w2_ref[...], preferred_element_type=jnp.float32)
    o_ref[...] = jnp.maximum(acc + b_ref[...], 0.0)


_BM = 1000


def _dense(x1, x2, w1, w2, b2d):
    return pl.pallas_call(
        _mm_body,
        grid=(B // _BM,),
        in_specs=[
            pl.BlockSpec((_BM, D), lambda i: (i, 0)),
            pl.BlockSpec((_BM, D), lambda i: (i, 0)),
            pl.BlockSpec((D, EMB), lambda i: (0, 0)),
            pl.BlockSpec((D, EMB), lambda i: (0, 0)),
            pl.BlockSpec((1, EMB), lambda i: (0, 0)),
        ],
        out_specs=pl.BlockSpec((_BM, EMB), lambda i: (i, 0)),
        out_shape=jax.ShapeDtypeStruct((B, EMB), jnp.float32),
    )(x1, x2, w1, w2, b2d)


def kernel(table, nodes, neigh_idx, W, b):
    nodes_i = nodes.astype(jnp.int32)
    neigh_i = neigh_idx.astype(jnp.int32)
    pad = BP - B
    nodes_p = jnp.concatenate([nodes_i, jnp.zeros((pad,), jnp.int32)])
    neigh_p = jnp.concatenate([neigh_i, jnp.zeros((pad, K), jnp.int32)])
    nodes_r = nodes_p.reshape(NW, NCH, C)
    neigh_r = jnp.transpose(neigh_p.reshape(NW, NCH, C, K), (0, 3, 1, 2))

    self_f, nsum = _sc_gather(nodes_r, neigh_r, table)

    w1 = W[:D]
    w2 = W[D:] * (1.0 / K)
    out = _dense(self_f[:B], nsum[:B], w1, w2, b.reshape(1, EMB))
    return out


# speedup vs baseline: 1.4592x; 1.0780x over previous
import functools

import jax
import jax.numpy as jnp
from jax import lax
from jax.experimental import pallas as pl
from jax.experimental.pallas import tpu as pltpu
from jax.experimental.pallas import tpu_sc as plsc

B = 10000
D = 256
DP = D // 2
K = 16
EMB = 256

NC = 2
NS = 16
NW = NC * NS
BPW = 320
BP = NW * BPW

BLKN = 8
BLKR = BLKN * K
NBLK = BPW // BLKN
NIT = NBLK // 2

SCH = 5
SC_C = 64
PGRP = DP // 16

_sc_mesh = plsc.VectorSubcoreMesh(core_axis_name="c", subcore_axis_name="s")
_HI = -65536


@functools.partial(
    pl.kernel,
    out_type=[
        jax.ShapeDtypeStruct((BP, D), jnp.float32),
        jax.ShapeDtypeStruct((BP, D), jnp.float32),
    ],
    mesh=_sc_mesh,
    scratch_types=[
        pltpu.VMEM((SCH, SC_C), jnp.int32),
        pltpu.VMEM((NBLK, BLKR), jnp.int32),
        pltpu.VMEM((SC_C, D), jnp.float32),
        pltpu.VMEM((SC_C, D), jnp.float32),
        pltpu.VMEM((BLKR, DP), jnp.int32),
        pltpu.VMEM((BLKR, DP), jnp.int32),
        pltpu.VMEM((BLKN, D), jnp.float32),
        pltpu.VMEM((BLKN, D), jnp.float32),
        pltpu.SemaphoreType.DMA,
        pltpu.SemaphoreType.DMA,
        pltpu.SemaphoreType.DMA,
        pltpu.SemaphoreType.DMA,
        pltpu.SemaphoreType.DMA,
        pltpu.SemaphoreType.DMA,
        pltpu.SemaphoreType.DMA,
        pltpu.SemaphoreType.DMA,
    ],
)
def _sc_gather(nodes_hbm, neigh_hbm, table_hbm, tpack_hbm, self_out, nsum_out,
               nodes_v, neigh_v, sv0, sv1, nb0, nb1, osum0, osum1,
               sem_n0, sem_n1, sem_o0, sem_o1,
               sem_si0, sem_si1, sem_so0, sem_so1):
    cid = lax.axis_index("c")
    sid = lax.axis_index("s")
    w = sid * NC + cid
    base = w * BPW

    pltpu.sync_copy(nodes_hbm.at[w], nodes_v)
    pltpu.sync_copy(neigh_hbm.at[w], neigh_v)

    pltpu.async_copy(tpack_hbm.at[neigh_v.at[0]], nb0, sem_n0)
    pltpu.async_copy(tpack_hbm.at[neigh_v.at[1]], nb1, sem_n1)

    sem_si = (sem_si0, sem_si1)
    sem_so = (sem_so0, sem_so1)
    sv = (sv0, sv1)
    h_in = [pltpu.async_copy(table_hbm.at[nodes_v.at[0]], sv0, sem_si0),
            pltpu.async_copy(table_hbm.at[nodes_v.at[1]], sv1, sem_si1)]
    for c in range(SCH):
        bsl = c % 2
        h_in[bsl].wait()
        ho = pltpu.async_copy(
            sv[bsl], self_out.at[pl.ds(base + c * SC_C, SC_C)], sem_so[bsl])
        if c + 2 < SCH:
            ho.wait()
            h_in[bsl] = pltpu.async_copy(
                table_hbm.at[nodes_v.at[c + 2]], sv[bsl], sem_si[bsl])

    def accum_block(buf, osum_b):
        shift16 = jnp.full((16,), 16, jnp.int32)
        himask = jnp.full((16,), _HI, jnp.int32)

        @plsc.parallel_loop(0, BLKN)
        def _(r):
            row = r * K
            for g in range(PGRP):
                sl = pl.ds(g * 16, 16)
                acc_a = jnp.zeros((16,), jnp.float32)
                acc_b = jnp.zeros((16,), jnp.float32)
                for j in range(K):
                    v = buf[row + j, sl]
                    acc_a = acc_a + lax.bitcast_convert_type(
                        lax.shift_left(v, shift16), jnp.float32)
                    acc_b = acc_b + lax.bitcast_convert_type(
                        lax.bitwise_and(v, himask), jnp.float32)
                osum_b[r, pl.ds(g * 32, 16)] = acc_a
                osum_b[r, pl.ds(g * 32 + 16, 16)] = acc_b

    def loop_body(i, _):
        row0 = base + i * (2 * BLKN)
        pltpu.make_async_copy(
            tpack_hbm.at[neigh_v.at[2 * i]], nb0, sem_n0).wait()

        @pl.when(i > 0)
        def _():
            pltpu.make_async_copy(
                osum0, nsum_out.at[pl.ds(row0 - 2 * BLKN, BLKN)],
                sem_o0).wait()

        accum_block(nb0, osum0)

        @pl.when(i < NIT - 1)
        def _():
            pltpu.async_copy(
                tpack_hbm.at[neigh_v.at[2 * i + 2]], nb0, sem_n0)

        pltpu.async_copy(osum0, nsum_out.at[pl.ds(row0, BLKN)], sem_o0)

        pltpu.make_async_copy(
            tpack_hbm.at[neigh_v.at[2 * i + 1]], nb1, sem_n1).wait()

        @pl.when(i > 0)
        def _():
            pltpu.make_async_copy(
                osum1, nsum_out.at[pl.ds(row0 - BLKN, BLKN)],
                sem_o1).wait()

        accum_block(nb1, osum1)

        @pl.when(i < NIT - 1)
        def _():
            pltpu.async_copy(
                tpack_hbm.at[neigh_v.at[2 * i + 3]], nb1, sem_n1)

        pltpu.async_copy(
            osum1, nsum_out.at[pl.ds(row0 + BLKN, BLKN)], sem_o1)
        return 0

    lax.fori_loop(0, NIT, loop_body, 0)

    last0 = base + (NIT - 1) * 2 * BLKN
    pltpu.make_async_copy(
        osum0, nsum_out.at[pl.ds(last0, BLKN)], sem_o0).wait()
    pltpu.make_async_copy(
        osum1, nsum_out.at[pl.ds(last0 + BLKN, BLKN)], sem_o1).wait()
    pltpu.make_async_copy(
        sv1, self_out.at[pl.ds(base + 3 * SC_C, SC_C)], sem_so1).wait()
    pltpu.make_async_copy(
        sv0, self_out.at[pl.ds(base + 4 * SC_C, SC_C)], sem_so0).wait()


def _mm_body(x1_ref, x2_ref, w1_ref, w2_ref, b_ref, o_ref):
    acc = jnp.dot(x1_ref[...], w1_ref[...], preferred_element_type=jnp.float32)
    acc = acc + jnp.dot(x2_ref[...], w2_ref[...], preferred_element_type=jnp.float32)
    o_ref[...] = jnp.maximum(acc + b_ref[...], 0.0)


_BM = 1000


def _dense(x1, x2, w1, w2, b2d):
    return pl.pallas_call(
        _mm_body,
        grid=(B // _BM,),
        in_specs=[
            pl.BlockSpec((_BM, D), lambda i: (i, 0)),
            pl.BlockSpec((_BM, D), lambda i: (i, 0)),
            pl.BlockSpec((D, EMB), lambda i: (0, 0)),
            pl.BlockSpec((D, EMB), lambda i: (0, 0)),
            pl.BlockSpec((1, EMB), lambda i: (0, 0)),
        ],
        out_specs=pl.BlockSpec((_BM, EMB), lambda i: (i, 0)),
        out_shape=jax.ShapeDtypeStruct((B, EMB), jnp.float32),
    )(x1, x2, w1, w2, b2d)


def kernel(table, nodes, neigh_idx, W, b):
    nodes_i = nodes.astype(jnp.int32)
    neigh_i = neigh_idx.astype(jnp.int32)
    pad = BP - B
    nodes_p = jnp.concatenate([nodes_i, jnp.zeros((pad,), jnp.int32)])
    neigh_p = jnp.concatenate([neigh_i, jnp.zeros((pad, K), jnp.int32)])
    nodes_r = nodes_p.reshape(NW, SCH, SC_C)
    neigh_r = neigh_p.reshape(NW, NBLK, BLKR)

    tb = table.astype(jnp.bfloat16)
    tp = tb.reshape(B, D // 32, 2, 16).transpose(0, 1, 3, 2)
    tpack = jax.lax.bitcast_convert_type(tp.reshape(B, DP, 2), jnp.int32)

    self_f, nsum = _sc_gather(nodes_r, neigh_r, table, tpack)

    w1 = W[:D]
    w2 = W[D:] * (1.0 / K)
    out = _dense(self_f[:B], nsum[:B], w1, w2, b.reshape(1, EMB))
    return out
